# Initial kernel scaffold; baseline (speedup 1.0000x reference)
import jax
import jax.numpy as jnp
from jax.experimental import pallas as pl


def kernel(x, edge_index, edge_attr, W1, We1, att1, proj1_w, proj1_b, W2, We2, att2, proj2_w, proj2_b):
    raise NotImplementedError("write your pallas kernel here")


# speedup vs baseline: 12.1012x; 12.1012x over previous
import functools

import jax
import jax.numpy as jnp
from jax import lax
from jax.experimental import pallas as pl
from jax.experimental.pallas import tpu as pltpu
from jax.experimental.pallas import tpu_sc as plsc

N = 10000
E = 320000
D_IN = 128
HID = 32
HEADS = 4
D_EDGE = 16
SLOPE = 0.2

C = 80
ZR = 200
NZCH = N // ZR


def _elu(v):
    return jnp.where(v > 0, v, jnp.exp(v) - 1.0)


def _tc1_node_body(x_ref, w_ref, a_ref, xw_ref, tab_ref):
    xw = jnp.dot(x_ref[...], w_ref[...], preferred_element_type=jnp.float32)
    xw_ref[...] = xw
    tab_ref[...] = jnp.dot(xw, a_ref[...], preferred_element_type=jnp.float32)


def _tc1_edge_body(ea_ref, ve_ref, ae1_ref, ae2_ref):
    v = jnp.dot(ea_ref[...], ve_ref[...], preferred_element_type=jnp.float32)
    ae1_ref[...] = v[:, :HEADS]
    ae2_ref[...] = v[:, HEADS:HEADS + 1]


def _tc2_body(p0_ref, p1_ref, pw_ref, pb_ref, w2_ref, a2_ref,
              xw2_ref, tab2_ref):
    p0 = p0_ref[...]
    p1 = p1_ref[...]
    feats = []
    for h in range(HEADS):
        p = p0 if h < 2 else p1
        loc = h % 2
        num = p[:, HID * loc:HID * loc + HID]
        den = p[:, 2 * HID + loc:2 * HID + loc + 1] + 1e-16
        feats.append(num / den)
    out1 = jnp.concatenate(feats, axis=1)
    h = _elu(jnp.dot(out1, pw_ref[...], preferred_element_type=jnp.float32)
             + pb_ref[...])
    h = _elu(h)
    xw2 = jnp.dot(h, w2_ref[...], preferred_element_type=jnp.float32)
    xw2_ref[...] = xw2
    tab2_ref[...] = jnp.dot(xw2, a2_ref[...], preferred_element_type=jnp.float32)


def _tc3_body(p0_ref, p1_ref, pw_ref, pb_ref, out_ref):
    acc = p0_ref[...] + p1_ref[...]
    out2 = acc[:, :HID] / (acc[:, HID:HID + 1] + 1e-16)
    out_ref[...] = _elu(
        jnp.dot(out2, pw_ref[...], preferred_element_type=jnp.float32)
        + pb_ref[...])


_MESH = plsc.VectorSubcoreMesh(core_axis_name="c", subcore_axis_name="s")


def _make_edge_pass(HH, edge_split):
    RW = HH * HID
    FV = RW // 16
    UW = FV + 1
    EPG = 16 // HH
    NG = C // EPG
    EPT = E // 16 if not edge_split else E // 32
    NCHUNK = EPT // C

    @functools.partial(
        pl.kernel, mesh=_MESH,
        out_type=jax.ShapeDtypeStruct((2 * N, 128), jnp.float32),
        scratch_types=[
            pltpu.VMEM((C,), jnp.int32),
            pltpu.VMEM((C,), jnp.int32),
            pltpu.VMEM((C,), jnp.int32),
            pltpu.VMEM((C,), jnp.int32),
            pltpu.VMEM((2, C), jnp.int32),
            pltpu.VMEM((2, C), jnp.float32),
            pltpu.VMEM((C * HH,), jnp.float32),
            pltpu.VMEM((C, 128), jnp.float32),
            pltpu.VMEM((C, 128), jnp.float32),
            pltpu.VMEM((ZR, 128), jnp.float32),
            pltpu.VMEM_SHARED((N, 128), jnp.float32),
            pltpu.SemaphoreType.DMA,
            pltpu.SemaphoreType.DMA,
        ],
    )
    def edge_pass(src_hbm, dst_hbm, ai_hbm, ae_hbm, xw_hbm, out_hbm,
                  srcv, dstv, srcvo, dstvo, idxb, gbufA, aev, rowsv, msgv,
                  zbuf, acc, sem0, sem1):
        cid = lax.axis_index("c")
        tid = lax.axis_index("s")
        iota = lax.iota(jnp.int32, 16)
        zeros16 = jnp.zeros((16,), jnp.float32)

        def zrow(t, _):
            r = t // 8
            k = t % 8
            zbuf.at[r][pl.ds(k * 16, 16)] = zeros16
            return _
        lax.fori_loop(0, ZR * 8, zrow, 0)

        def mpad(j, _):
            for s in range(UW, 8):
                msgv.at[j][pl.ds(s * 16, 16)] = zeros16
            return _
        lax.fori_loop(0, C, mpad, 0)

        for k in range(-(-NZCH // 16)):
            zc = tid + 16 * k
            @pl.when(zc < NZCH)
            def _():
                pltpu.sync_copy(zbuf, acc.at[pl.ds(zc * ZR, ZR)])
        plsc.subcore_barrier()

        def chunk(ch, _):
            if edge_split:
                base = (tid * 2 + cid) * EPT + ch * C
            else:
                base = tid * EPT + ch * C
            if edge_split:
                aoff = base * HH
            else:
                aoff = cid * (2 * E) + base * HH
            pltpu.sync_copy(src_hbm.at[pl.ds(base, C)], srcv)
            pltpu.sync_copy(dst_hbm.at[pl.ds(base, C)], dstv)
            pltpu.sync_copy(ae_hbm.at[pl.ds(aoff, C * HH)], aev)

            if edge_split:
                cp_ai = pltpu.async_copy(ai_hbm.at[dstv], gbufA.at[0], sem0)
                cp_rows = pltpu.async_copy(xw_hbm.at[srcv], rowsv, sem1)
            else:
                def offs(t, _):
                    sl = pl.ds(t * 16, 16)
                    srcvo[sl] = srcv[sl] + cid * N
                    dstvo[sl] = dstv[sl] * 2 + cid * (2 * N)
                    return _
                lax.fori_loop(0, C // 16, offs, 0)
                for w in range(2 * C // 16):
                    dvo = dstvo[pl.ds((w // 2) * 16, 16)]
                    rep = dvo.at[8 * (w % 2) + (iota >> 1)].get(
                        mode="promise_in_bounds")
                    val = rep + (iota & 1)
                    idxb.at[(16 * w) // C][pl.ds((16 * w) % C, 16)] = val
                cp_ai0 = pltpu.async_copy(ai_hbm.at[idxb.at[0]],
                                          gbufA.at[0], sem0)
                cp_rows = pltpu.async_copy(xw_hbm.at[srcvo], rowsv, sem1)
                cp_ai0.wait()
                cp_ai = pltpu.async_copy(ai_hbm.at[idxb.at[1]],
                                         gbufA.at[1], sem0)
            cp_ai.wait()
            cp_rows.wait()

            def group(r, gw):
                g = r * (NG // 2) + gw if not edge_split else gw
                av = gbufA[r, pl.ds(16 * gw, 16)] if not edge_split \
                    else gbufA[0, pl.ds(16 * gw, 16)]
                ev = aev[pl.ds(16 * g, 16)]
                sv = av + ev
                for m in range(EPG):
                    j = g * EPG + m
                    rot = sv.at[(m * HH + iota) & 15].get(
                        mode="promise_in_bounds")
                    tailv = rowsv[j, pl.ds(RW, 16)]
                    s = rot + tailv
                    e = jnp.where(s > 0, s, SLOPE * s)
                    ex = jnp.exp(e)
                    for fs in range(FV):
                        b = ex.at[jnp.broadcast_to(
                            jnp.int32((fs * 16) // HID), (16,))].get(
                            mode="promise_in_bounds")
                        msgv.at[j][pl.ds(fs * 16, 16)] = \
                            rowsv[j, pl.ds(fs * 16, 16)] * b
                    msgv.at[j][pl.ds(RW, 16)] = jnp.where(iota < HH, ex, 0.0)

            if edge_split:
                def g_body(gw, _):
                    group(0, gw)
                    return _
                lax.fori_loop(0, NG, g_body, 0)
            else:
                for r in range(2):
                    def g_body(gw, _, _r=r):
                        group(_r, gw)
                        return _
                    lax.fori_loop(0, NG // 2, g_body, 0)

            pltpu.sync_copy(msgv, acc.at[dstv], add=True)
            return _
        lax.fori_loop(0, NCHUNK, chunk, 0)

        plsc.subcore_barrier()
        for k in range(-(-NZCH // 16)):
            zc = tid + 16 * k
            @pl.when(zc < NZCH)
            def _():
                r0 = zc * ZR
                pltpu.sync_copy(acc.at[pl.ds(r0, ZR)], zbuf)
                pltpu.sync_copy(zbuf, out_hbm.at[pl.ds(cid * N + r0, ZR)])

    return edge_pass


_edge_pass_l1 = _make_edge_pass(2, edge_split=False)
_edge_pass_l2 = _make_edge_pass(1, edge_split=True)


def kernel(x, edge_index, edge_attr, W1, We1, att1, proj1_w, proj1_b,
           W2, We2, att2, proj2_w, proj2_b):
    f32 = jnp.float32
    src = edge_index[0]
    dst = edge_index[1]

    W1cat = jnp.concatenate([W1[h] for h in range(HEADS)], axis=1)
    att = att1[:, :, 0]
    Ai = jnp.zeros((D_IN, HEADS), f32)
    Aj = jnp.zeros((D_IN, HEADS), f32)
    for h in range(HEADS):
        Ai = Ai.at[h * HID:(h + 1) * HID, h].set(att[h, :HID])
        Aj = Aj.at[h * HID:(h + 1) * HID, h].set(att[h, HID:2 * HID])
    A1 = jnp.concatenate([Ai, Aj], axis=1)
    Ve = jnp.stack([We1[h] @ att[h, 2 * HID:] for h in range(HEADS)], axis=1)
    ve2 = We2 @ att2[2 * HID:, 0]
    VeAll = jnp.concatenate(
        [Ve, ve2[:, None], jnp.zeros((D_EDGE, 3), f32)], axis=1)
    A2 = jnp.concatenate(
        [att2[:HID, :1], att2[HID:2 * HID, :1]], axis=1)

    BN = 1000
    nb = N // BN
    xw, tab1 = pl.pallas_call(
        _tc1_node_body,
        grid=(nb,),
        in_specs=[pl.BlockSpec((BN, D_IN), lambda i: (i, 0)),
                  pl.BlockSpec((D_IN, D_IN), lambda i: (0, 0)),
                  pl.BlockSpec((D_IN, 8), lambda i: (0, 0))],
        out_specs=[pl.BlockSpec((BN, D_IN), lambda i: (i, 0)),
                   pl.BlockSpec((BN, 8), lambda i: (i, 0))],
        out_shape=[jax.ShapeDtypeStruct((N, D_IN), f32),
                   jax.ShapeDtypeStruct((N, 8), f32)],
    )(x, W1cat, A1)

    BE = 4000
    ae1, ae2 = pl.pallas_call(
        _tc1_edge_body,
        grid=(E // BE,),
        in_specs=[pl.BlockSpec((BE, D_EDGE), lambda i: (i, 0)),
                  pl.BlockSpec((D_EDGE, 8), lambda i: (0, 0))],
        out_specs=[pl.BlockSpec((BE, HEADS), lambda i: (i, 0)),
                   pl.BlockSpec((BE, 1), lambda i: (i, 0))],
        out_shape=[jax.ShapeDtypeStruct((E, HEADS), f32),
                   jax.ShapeDtypeStruct((E, 1), f32)],
    )(edge_attr, VeAll)

    pad62 = jnp.zeros((N, 62), f32)
    xwext1 = jnp.concatenate(
        [jnp.concatenate([xw[:, 64 * c:64 * c + 64],
                          tab1[:, 4 + 2 * c:4 + 2 * c + 2], pad62], axis=1)
         for c in range(2)], axis=0)
    ai1 = tab1[:, 0:4].reshape(N, 2, 2).transpose(1, 0, 2).reshape(-1)
    ae1f = ae1.reshape(E, 2, 2).transpose(1, 0, 2).reshape(-1)

    part1 = _edge_pass_l1(src, dst, ai1, ae1f, xwext1)

    xw2, tab2 = pl.pallas_call(
        _tc2_body,
        grid=(nb,),
        in_specs=[pl.BlockSpec((BN, 128), lambda i: (i, 0)),
                  pl.BlockSpec((BN, 128), lambda i, _nb=nb: (i + _nb, 0)),
                  pl.BlockSpec((D_IN, D_IN), lambda i: (0, 0)),
                  pl.BlockSpec((1, D_IN), lambda i: (0, 0)),
                  pl.BlockSpec((D_IN, HID), lambda i: (0, 0)),
                  pl.BlockSpec((HID, 2), lambda i: (0, 0))],
        out_specs=[pl.BlockSpec((BN, HID), lambda i: (i, 0)),
                   pl.BlockSpec((BN, 2), lambda i: (i, 0))],
        out_shape=[jax.ShapeDtypeStruct((N, HID), f32),
                   jax.ShapeDtypeStruct((N, 2), f32)],
    )(part1, part1, proj1_w, proj1_b[None, :], W2, A2)

    xwext2 = jnp.concatenate(
        [xw2, tab2[:, 1:2], jnp.zeros((N, 95), f32)], axis=1)
    ai2 = tab2[:, 0]

    part2 = _edge_pass_l2(src, dst, ai2, ae2.reshape(-1), xwext2)

    importance = pl.pallas_call(
        _tc3_body,
        grid=(nb,),
        in_specs=[pl.BlockSpec((BN, 128), lambda i: (i, 0)),
                  pl.BlockSpec((BN, 128), lambda i, _nb=nb: (i + _nb, 0)),
                  pl.BlockSpec((HID, 1), lambda i: (0, 0)),
                  pl.BlockSpec((1, 1), lambda i: (0, 0))],
        out_specs=pl.BlockSpec((BN, 1), lambda i: (i, 0)),
        out_shape=jax.ShapeDtypeStruct((N, 1), f32),
    )(part2, part2, proj2_w, proj2_b[None, :])

    return importance


# speedup vs baseline: 21.0980x; 1.7435x over previous
import functools

import jax
import jax.numpy as jnp
from jax import lax
from jax.experimental import pallas as pl
from jax.experimental.pallas import tpu as pltpu
from jax.experimental.pallas import tpu_sc as plsc

N = 10000
E = 320000
D_IN = 128
HID = 32
HEADS = 4
D_EDGE = 16
SLOPE = 0.2

C = 80
ZR = 40
NZCH = N // ZR


def _elu(v):
    return jnp.where(v > 0, v, jnp.exp(v) - 1.0)


def _tc1_node_body(x_ref, w_ref, a_ref, xwext_ref, ai_ref):
    c = pl.program_id(0)
    xw = jnp.dot(x_ref[...], w_ref[...], preferred_element_type=jnp.float32)
    tab = jnp.dot(xw, a_ref[...], preferred_element_type=jnp.float32)
    bn = xw.shape[0]
    xwc = jnp.where(c == 0, xw[:, 0:64], xw[:, 64:128])
    ajc = jnp.where(c == 0, tab[:, 4:6], tab[:, 6:8])
    xwext_ref[...] = jnp.concatenate(
        [xwc, ajc, jnp.zeros((bn, 62), jnp.float32)], axis=1)
    ai_ref[...] = jnp.where(c == 0, tab[:, 0:2], tab[:, 2:4])[None]


def _tc1_edge_body(ea_ref, ve_ref, ae1_ref, ae2_ref):
    c = pl.program_id(0)
    v = jnp.dot(ea_ref[...], ve_ref[...], preferred_element_type=jnp.float32)
    ae1_ref[...] = jnp.where(c == 0, v[:, 0:2], v[:, 2:4])[None]
    ae2_ref[...] = v[:, HEADS:HEADS + 1]


def _tc2_body(p0_ref, p1_ref, pw_ref, pb_ref, w2_ref, a2_ref,
              xw2ext_ref, ai2_ref):
    p0 = p0_ref[...]
    p1 = p1_ref[...]
    feats = []
    for h in range(HEADS):
        p = p0 if h < 2 else p1
        loc = h % 2
        num = p[:, HID * loc:HID * loc + HID]
        den = p[:, 2 * HID + loc:2 * HID + loc + 1] + 1e-16
        feats.append(num / den)
    out1 = jnp.concatenate(feats, axis=1)
    h = _elu(jnp.dot(out1, pw_ref[...], preferred_element_type=jnp.float32)
             + pb_ref[...])
    h = _elu(h)
    xw2 = jnp.dot(h, w2_ref[...], preferred_element_type=jnp.float32)
    tab2 = jnp.dot(xw2, a2_ref[...], preferred_element_type=jnp.float32)
    bn = xw2.shape[0]
    xw2ext_ref[...] = jnp.concatenate(
        [xw2, tab2[:, 1:2], jnp.zeros((bn, 95), jnp.float32)], axis=1)
    ai2_ref[...] = tab2[:, 0:1]


def _tc3_body(p0_ref, p1_ref, pw_ref, pb_ref, out_ref):
    acc = p0_ref[...] + p1_ref[...]
    out2 = acc[:, :HID] / (acc[:, HID:HID + 1] + 1e-16)
    out_ref[...] = _elu(
        jnp.dot(out2, pw_ref[...], preferred_element_type=jnp.float32)
        + pb_ref[...])


_MESH = plsc.VectorSubcoreMesh(core_axis_name="c", subcore_axis_name="s")


def _make_edge_pass(HH, edge_split):
    RW = HH * HID
    FV = RW // 16
    UW = FV + 1
    EPG = 16 // HH
    NG = C // EPG
    EPT = E // 16 if not edge_split else E // 32
    NCH = EPT // C
    NP = (NCH + 1) // 2

    @functools.partial(
        pl.kernel, mesh=_MESH,
        out_type=jax.ShapeDtypeStruct((2 * N, 128), jnp.float32),
        scratch_types=(
            [pltpu.VMEM((C,), jnp.int32) for _ in range(10)]
            + [pltpu.VMEM((2, C), jnp.int32) for _ in range(2)]
            + [pltpu.VMEM((2, C), jnp.float32) for _ in range(2)]
            + [pltpu.VMEM((C * HH,), jnp.float32) for _ in range(2)]
            + [pltpu.VMEM((C, 128), jnp.float32) for _ in range(4)]
            + [pltpu.VMEM((ZR, 128), jnp.float32)]
            + [pltpu.VMEM_SHARED((N, 128), jnp.float32)]
            + [pltpu.SemaphoreType.DMA for _ in range(12)]
        ),
    )
    def edge_pass(src_hbm, dst_hbm, ai_hbm, ae_hbm, xw_hbm, out_hbm,
                  srcv0, srcv1, dstv0, dstv1, dscat0, dscat1,
                  srcvo0, srcvo1, dstvo0, dstvo1,
                  idxb0, idxb1, gbufA0, gbufA1, aev0, aev1,
                  rowsv0, rowsv1, msgv0, msgv1, zbuf, acc,
                  lsrc0, lsrc1, ldst0, ldst1, grow0, grow1,
                  gai0, gai1, gae0, gae1, scat0, scat1):
        SRC = [srcv0, srcv1]
        DST = [dstv0, dstv1]
        DSC = [dscat0, dscat1]
        SRCO = [srcvo0, srcvo1] if not edge_split else SRC
        DSTO = [dstvo0, dstvo1]
        IDXB = [idxb0, idxb1]
        GA = [gbufA0, gbufA1]
        AEV = [aev0, aev1]
        ROWS = [rowsv0, rowsv1]
        MSG = [msgv0, msgv1]
        LSRC = [lsrc0, lsrc1]
        LDST = [ldst0, ldst1]
        GROW = [grow0, grow1]
        GAI = [gai0, gai1]
        GAE = [gae0, gae1]
        SCAT = [scat0, scat1]

        cid = lax.axis_index("c")
        tid = lax.axis_index("s")
        iota = lax.iota(jnp.int32, 16)
        zeros16 = jnp.zeros((16,), jnp.float32)

        def chunk_base(ch):
            if edge_split:
                return (tid * 2 + cid) * EPT + ch * C
            return tid * EPT + ch * C

        def zrow(t, _):
            r = t // 8
            k = t % 8
            zbuf.at[r][pl.ds(k * 16, 16)] = zeros16
            return _
        lax.fori_loop(0, ZR * 8, zrow, 0)

        def mpad(j, _):
            for p in range(2):
                for s in range(UW, 8):
                    MSG[p].at[j][pl.ds(s * 16, 16)] = zeros16
            return _
        lax.fori_loop(0, C, mpad, 0)

        for k in range(-(-NZCH // 16)):
            zc = tid + 16 * k
            @pl.when(zc < NZCH)
            def _():
                pltpu.sync_copy(zbuf, acc.at[pl.ds(zc * ZR, ZR)])
        plsc.subcore_barrier()

        def fire_linear(ch, p):
            @pl.when(ch < NCH)
            def _():
                base = chunk_base(ch)
                pltpu.async_copy(src_hbm.at[pl.ds(base, C)], SRC[p], LSRC[p])
                pltpu.async_copy(dst_hbm.at[pl.ds(base, C)], DST[p], LDST[p])

        def stage_a(ch, p):
            @pl.when(ch < NCH)
            def _():
                pltpu.make_async_copy(
                    src_hbm.at[pl.ds(0, C)], SRC[p], LSRC[p]).wait()
                pltpu.make_async_copy(
                    dst_hbm.at[pl.ds(0, C)], DST[p], LDST[p]).wait()
                base = chunk_base(ch)
                if edge_split:
                    aoff = base * HH
                else:
                    aoff = cid * (2 * E) + base * HH
                pltpu.async_copy(ae_hbm.at[pl.ds(aoff, C * HH)],
                                 AEV[p], GAE[p])
                if edge_split:
                    pltpu.async_copy(ai_hbm.at[DST[p]], GA[p].at[0], GAI[p])
                    pltpu.async_copy(xw_hbm.at[SRC[p]], ROWS[p], GROW[p])
                else:
                    for t in range(C // 16):
                        sl = pl.ds(t * 16, 16)
                        SRCO[p][sl] = SRC[p][sl] + cid * N
                        DSTO[p][sl] = DST[p][sl] * 2 + cid * (2 * N)
                    for w in range(2 * C // 16):
                        dvo = DSTO[p][pl.ds((w // 2) * 16, 16)]
                        rep = dvo.at[8 * (w % 2) + (iota >> 1)].get(
                            mode="promise_in_bounds")
                        IDXB[p].at[(16 * w) // C][pl.ds((16 * w) % C, 16)] = \
                            rep + (iota & 1)
                    pltpu.async_copy(ai_hbm.at[IDXB[p].at[0]],
                                     GA[p].at[0], GAI[p])
                    pltpu.async_copy(ai_hbm.at[IDXB[p].at[1]],
                                     GA[p].at[1], GAI[p])
                    pltpu.async_copy(xw_hbm.at[SRCO[p]], ROWS[p], GROW[p])

        def stage_b(ch, p):
            pltpu.make_async_copy(
                xw_hbm.at[SRCO[p]], ROWS[p], GROW[p]).wait()
            if edge_split:
                pltpu.make_async_copy(
                    ai_hbm.at[DST[p]], GA[p].at[0], GAI[p]).wait()
            else:
                pltpu.make_async_copy(
                    ai_hbm.at[IDXB[p].at[0]], GA[p].at[0], GAI[p]).wait()
                pltpu.make_async_copy(
                    ai_hbm.at[IDXB[p].at[1]], GA[p].at[1], GAI[p]).wait()
            pltpu.make_async_copy(
                ae_hbm.at[pl.ds(0, C * HH)], AEV[p], GAE[p]).wait()
            @pl.when(ch >= 2)
            def _():
                pltpu.make_async_copy(MSG[p], acc.at[DSC[p]], SCAT[p]).wait()
            for t in range(C // 16):
                sl = pl.ds(t * 16, 16)
                DSC[p][sl] = DST[p][sl]
            fire_linear(ch + 2, p)

            def group(r, gw, g):
                av = GA[p][r, pl.ds(16 * gw, 16)]
                ev = AEV[p][pl.ds(16 * g, 16)]
                sv = av + ev
                for m in range(EPG):
                    j = g * EPG + m
                    rot = sv.at[(m * HH + iota) & 15].get(
                        mode="promise_in_bounds")
                    tailv = ROWS[p][j, pl.ds(RW, 16)]
                    s = rot + tailv
                    e = jnp.where(s > 0, s, SLOPE * s)
                    ex = jnp.exp(e)
                    for fs in range(FV):
                        b = ex.at[jnp.broadcast_to(
                            jnp.int32((fs * 16) // HID), (16,))].get(
                            mode="promise_in_bounds")
                        MSG[p].at[j][pl.ds(fs * 16, 16)] = \
                            ROWS[p][j, pl.ds(fs * 16, 16)] * b
                    MSG[p].at[j][pl.ds(RW, 16)] = jnp.where(
                        iota < HH, ex, 0.0)

            if edge_split:
                def g_body(gw, _):
                    group(0, gw, gw)
                    return _
                lax.fori_loop(0, NG, g_body, 0)
            else:
                for r in range(2):
                    def g_body(gw, _, _r=r):
                        group(_r, gw, _r * (NG // 2) + gw)
                        return _
                    lax.fori_loop(0, NG // 2, g_body, 0)

            pltpu.async_copy(MSG[p], acc.at[DSC[p]], SCAT[p], add=True)

        fire_linear(0, 0)
        fire_linear(1, 1)
        stage_a(0, 0)

        def pair(k, carry):
            ch0 = 2 * k
            stage_a(ch0 + 1, 1)
            stage_b(ch0, 0)
            @pl.when(ch0 + 1 < NCH)
            def _odd():
                stage_a(ch0 + 2, 0)
                stage_b(ch0 + 1, 1)
            return carry
        lax.fori_loop(0, NP, pair, 0)

        for p in range(2):
            pltpu.make_async_copy(MSG[p], acc.at[DSC[p]], SCAT[p]).wait()

        plsc.subcore_barrier()
        for k in range(-(-NZCH // 16)):
            zc = tid + 16 * k
            @pl.when(zc < NZCH)
            def _():
                r0 = zc * ZR
                pltpu.sync_copy(acc.at[pl.ds(r0, ZR)], zbuf)
                pltpu.sync_copy(zbuf, out_hbm.at[pl.ds(cid * N + r0, ZR)])

    return edge_pass


_edge_pass_l1 = _make_edge_pass(2, edge_split=False)
_edge_pass_l2 = _make_edge_pass(1, edge_split=True)


def kernel(x, edge_index, edge_attr, W1, We1, att1, proj1_w, proj1_b,
           W2, We2, att2, proj2_w, proj2_b):
    f32 = jnp.float32
    src = edge_index[0]
    dst = edge_index[1]

    W1cat = jnp.concatenate([W1[h] for h in range(HEADS)], axis=1)
    att = att1[:, :, 0]
    Ai = jnp.zeros((D_IN, HEADS), f32)
    Aj = jnp.zeros((D_IN, HEADS), f32)
    for h in range(HEADS):
        Ai = Ai.at[h * HID:(h + 1) * HID, h].set(att[h, :HID])
        Aj = Aj.at[h * HID:(h + 1) * HID, h].set(att[h, HID:2 * HID])
    A1 = jnp.concatenate([Ai, Aj], axis=1)
    Ve = jnp.stack([We1[h] @ att[h, 2 * HID:] for h in range(HEADS)], axis=1)
    ve2 = We2 @ att2[2 * HID:, 0]
    VeAll = jnp.concatenate(
        [Ve, ve2[:, None], jnp.zeros((D_EDGE, 3), f32)], axis=1)
    A2 = jnp.concatenate(
        [att2[:HID, :1], att2[HID:2 * HID, :1]], axis=1)

    BN = 1000
    nb = N // BN
    xwext1, ai3 = pl.pallas_call(
        _tc1_node_body,
        grid=(2, nb),
        in_specs=[pl.BlockSpec((BN, D_IN), lambda c, i: (i, 0)),
                  pl.BlockSpec((D_IN, D_IN), lambda c, i: (0, 0)),
                  pl.BlockSpec((D_IN, 8), lambda c, i: (0, 0))],
        out_specs=[pl.BlockSpec((BN, 128), lambda c, i, _nb=nb: (c * _nb + i, 0)),
                   pl.BlockSpec((1, BN, 2), lambda c, i: (c, i, 0))],
        out_shape=[jax.ShapeDtypeStruct((2 * N, 128), f32),
                   jax.ShapeDtypeStruct((2, N, 2), f32)],
    )(x, W1cat, A1)

    BE = 4000
    ae13, ae2 = pl.pallas_call(
        _tc1_edge_body,
        grid=(2, E // BE),
        in_specs=[pl.BlockSpec((BE, D_EDGE), lambda c, i: (i, 0)),
                  pl.BlockSpec((D_EDGE, 8), lambda c, i: (0, 0))],
        out_specs=[pl.BlockSpec((1, BE, 2), lambda c, i: (c, i, 0)),
                   pl.BlockSpec((BE, 1), lambda c, i: (i, 0))],
        out_shape=[jax.ShapeDtypeStruct((2, E, 2), f32),
                   jax.ShapeDtypeStruct((E, 1), f32)],
    )(edge_attr, VeAll)

    part1 = _edge_pass_l1(src, dst, ai3.reshape(-1), ae13.reshape(-1),
                          xwext1)

    xw2ext, ai2 = pl.pallas_call(
        _tc2_body,
        grid=(nb,),
        in_specs=[pl.BlockSpec((BN, 128), lambda i: (i, 0)),
                  pl.BlockSpec((BN, 128), lambda i, _nb=nb: (i + _nb, 0)),
                  pl.BlockSpec((D_IN, D_IN), lambda i: (0, 0)),
                  pl.BlockSpec((1, D_IN), lambda i: (0, 0)),
                  pl.BlockSpec((D_IN, HID), lambda i: (0, 0)),
                  pl.BlockSpec((HID, 2), lambda i: (0, 0))],
        out_specs=[pl.BlockSpec((BN, 128), lambda i: (i, 0)),
                   pl.BlockSpec((BN, 1), lambda i: (i, 0))],
        out_shape=[jax.ShapeDtypeStruct((N, 128), f32),
                   jax.ShapeDtypeStruct((N, 1), f32)],
    )(part1, part1, proj1_w, proj1_b[None, :], W2, A2)

    part2 = _edge_pass_l2(src, dst, ai2.reshape(-1), ae2.reshape(-1),
                          xw2ext)

    importance = pl.pallas_call(
        _tc3_body,
        grid=(nb,),
        in_specs=[pl.BlockSpec((BN, 128), lambda i: (i, 0)),
                  pl.BlockSpec((BN, 128), lambda i, _nb=nb: (i + _nb, 0)),
                  pl.BlockSpec((HID, 1), lambda i: (0, 0)),
                  pl.BlockSpec((1, 1), lambda i: (0, 0))],
        out_specs=pl.BlockSpec((BN, 1), lambda i: (i, 0)),
        out_shape=jax.ShapeDtypeStruct((N, 1), f32),
    )(part2, part2, proj2_w, proj2_b[None, :])

    return importance


# speedup vs baseline: 30.1020x; 1.4268x over previous
import functools

import jax
import jax.numpy as jnp
from jax import lax
from jax.experimental import pallas as pl
from jax.experimental.pallas import tpu as pltpu
from jax.experimental.pallas import tpu_sc as plsc

N = 10000
E = 320000
D_IN = 128
HID = 32
HEADS = 4
D_EDGE = 16
SLOPE = 0.2

C = 80
ZR = 40
NZCH = N // ZR


def _elu(v):
    return jnp.where(v > 0, v, jnp.exp(v) - 1.0)


def _tc1_node_body(x_ref, w_ref, a_ref, xwext_ref, ai_ref):
    xw = jnp.dot(x_ref[...], w_ref[...], preferred_element_type=jnp.float32)
    tab = jnp.dot(xw, a_ref[...], preferred_element_type=jnp.float32)
    bn = xw.shape[0]
    z = jnp.zeros((bn, 62), jnp.float32)
    xwext_ref[...] = jnp.concatenate(
        [jnp.concatenate([xw[:, 0:64], tab[:, 4:6], z], axis=1),
         jnp.concatenate([xw[:, 64:128], tab[:, 6:8], z], axis=1)], axis=0)
    ai_ref[...] = lax.dot_general(
        a_ref[...], xw, (((0,), (1,)), ((), ())),
        preferred_element_type=jnp.float32)


def _tc1_edge_body(ea_ref, ve_ref, ae8_ref):
    ae8_ref[...] = lax.dot_general(
        ve_ref[...], ea_ref[...], (((0,), (1,)), ((), ())),
        preferred_element_type=jnp.float32)


def _tc2_body(p0_ref, p1_ref, pw_ref, pb_ref, w2_ref, a2_ref,
              xw2ext_ref, ai2_ref):
    p0 = p0_ref[...]
    p1 = p1_ref[...]
    feats = []
    for h in range(HEADS):
        p = p0 if h < 2 else p1
        loc = h % 2
        num = p[:, HID * loc:HID * loc + HID]
        den = p[:, 2 * HID + loc:2 * HID + loc + 1] + 1e-16
        feats.append(num / den)
    out1 = jnp.concatenate(feats, axis=1)
    h = _elu(jnp.dot(out1, pw_ref[...], preferred_element_type=jnp.float32)
             + pb_ref[...])
    h = _elu(h)
    xw2 = jnp.dot(h, w2_ref[...], preferred_element_type=jnp.float32)
    tab2 = jnp.dot(xw2, a2_ref[...], preferred_element_type=jnp.float32)
    bn = xw2.shape[0]
    xw2ext_ref[...] = jnp.concatenate(
        [xw2, tab2[:, 1:2], jnp.zeros((bn, 95), jnp.float32)], axis=1)
    ai2_ref[...] = tab2[:, 0:1]


def _tc3_body(p0_ref, p1_ref, pw_ref, pb_ref, out_ref):
    acc = p0_ref[...] + p1_ref[...]
    out2 = acc[:, :HID] / (acc[:, HID:HID + 1] + 1e-16)
    out_ref[...] = _elu(
        jnp.dot(out2, pw_ref[...], preferred_element_type=jnp.float32)
        + pb_ref[...])


_MESH = plsc.VectorSubcoreMesh(core_axis_name="c", subcore_axis_name="s")


def _make_edge_pass(HH, edge_split):
    RW = HH * HID
    FV = RW // 16
    UW = FV + 1
    EPG = 16 // HH
    NG = C // EPG
    EPT = E // 16 if not edge_split else E // 32
    NCH = EPT // C
    NP = (NCH + 1) // 2

    @functools.partial(
        pl.kernel, mesh=_MESH,
        out_type=jax.ShapeDtypeStruct((2 * N, 128), jnp.float32),
        scratch_types=(
            [pltpu.VMEM((C,), jnp.int32) for _ in range(10)]
            + [pltpu.VMEM((2, C), jnp.int32) for _ in range(2)]
            + [pltpu.VMEM((2, C), jnp.float32) for _ in range(2)]
            + [pltpu.VMEM((C * HH,), jnp.float32) for _ in range(2)]
            + [pltpu.VMEM((C, 128), jnp.float32) for _ in range(4)]
            + [pltpu.VMEM((ZR, 128), jnp.float32)]
            + [pltpu.VMEM_SHARED((N, 128), jnp.float32)]
            + [pltpu.SemaphoreType.DMA for _ in range(12)]
        ),
    )
    def edge_pass(src_hbm, dst_hbm, ai_hbm, ae_hbm, xw_hbm, out_hbm,
                  srcv0, srcv1, dstv0, dstv1, dscat0, dscat1,
                  srcvo0, srcvo1, dstvo0, dstvo1,
                  idxb0, idxb1, gbufA0, gbufA1, aev0, aev1,
                  rowsv0, rowsv1, msgv0, msgv1, zbuf, acc,
                  lsrc0, lsrc1, ldst0, ldst1, grow0, grow1,
                  gai0, gai1, gae0, gae1, scat0, scat1):
        SRC = [srcv0, srcv1]
        DST = [dstv0, dstv1]
        DSC = [dscat0, dscat1]
        SRCO = [srcvo0, srcvo1] if not edge_split else SRC
        DSTO = [dstvo0, dstvo1]
        IDXB = [idxb0, idxb1]
        GA = [gbufA0, gbufA1]
        AEV = [aev0, aev1]
        ROWS = [rowsv0, rowsv1]
        MSG = [msgv0, msgv1]
        LSRC = [lsrc0, lsrc1]
        LDST = [ldst0, ldst1]
        GROW = [grow0, grow1]
        GAI = [gai0, gai1]
        GAE = [gae0, gae1]
        SCAT = [scat0, scat1]

        cid = lax.axis_index("c")
        tid = lax.axis_index("s")
        iota = lax.iota(jnp.int32, 16)
        zeros16 = jnp.zeros((16,), jnp.float32)

        def chunk_base(ch):
            if edge_split:
                return (tid * 2 + cid) * EPT + ch * C
            return tid * EPT + ch * C

        def zrow(t, _):
            r = t // 8
            k = t % 8
            zbuf.at[r][pl.ds(k * 16, 16)] = zeros16
            return _
        lax.fori_loop(0, ZR * 8, zrow, 0)

        def mpad(j, _):
            for p in range(2):
                for s in range(UW, 8):
                    MSG[p].at[j][pl.ds(s * 16, 16)] = zeros16
            return _
        lax.fori_loop(0, C, mpad, 0)

        for k in range(-(-NZCH // 16)):
            zc = tid + 16 * k
            @pl.when(zc < NZCH)
            def _():
                pltpu.sync_copy(zbuf, acc.at[pl.ds(zc * ZR, ZR)])
        plsc.subcore_barrier()

        def fire_linear(ch, p):
            @pl.when(ch < NCH)
            def _():
                base = chunk_base(ch)
                pltpu.async_copy(src_hbm.at[pl.ds(base, C)], SRC[p], LSRC[p])
                pltpu.async_copy(dst_hbm.at[pl.ds(base, C)], DST[p], LDST[p])

        def stage_a(ch, p):
            @pl.when(ch < NCH)
            def _():
                pltpu.make_async_copy(
                    src_hbm.at[pl.ds(0, C)], SRC[p], LSRC[p]).wait()
                pltpu.make_async_copy(
                    dst_hbm.at[pl.ds(0, C)], DST[p], LDST[p]).wait()
                base = chunk_base(ch)
                if edge_split:
                    pltpu.async_copy(ae_hbm.at[pl.ds(4 * E + base, C)],
                                     AEV[p], GAE[p])
                    pltpu.async_copy(ai_hbm.at[DST[p]], GA[p].at[0], GAI[p])
                    pltpu.async_copy(xw_hbm.at[SRC[p]], ROWS[p], GROW[p])
                else:
                    aoff = cid * (2 * E) + base
                    pltpu.async_copy(ae_hbm.at[pl.ds(aoff, C)],
                                     AEV[p].at[pl.ds(0, C)], GAE[p])
                    pltpu.async_copy(ae_hbm.at[pl.ds(aoff + E, C)],
                                     AEV[p].at[pl.ds(C, C)], GAE[p])
                    for t in range(C // 16):
                        sl = pl.ds(t * 16, 16)
                        SRCO[p][sl] = SRC[p][sl] + cid * N
                        IDXB[p].at[0][sl] = DST[p][sl] + cid * (2 * N)
                        IDXB[p].at[1][sl] = DST[p][sl] + cid * (2 * N) + N
                    pltpu.async_copy(ai_hbm.at[IDXB[p].at[0]],
                                     GA[p].at[0], GAI[p])
                    pltpu.async_copy(ai_hbm.at[IDXB[p].at[1]],
                                     GA[p].at[1], GAI[p])
                    pltpu.async_copy(xw_hbm.at[SRCO[p]], ROWS[p], GROW[p])

        def stage_b(ch, p):
            pltpu.make_async_copy(
                xw_hbm.at[SRCO[p]], ROWS[p], GROW[p]).wait()
            if edge_split:
                pltpu.make_async_copy(
                    ai_hbm.at[DST[p]], GA[p].at[0], GAI[p]).wait()
            else:
                pltpu.make_async_copy(
                    ai_hbm.at[IDXB[p].at[0]], GA[p].at[0], GAI[p]).wait()
                pltpu.make_async_copy(
                    ai_hbm.at[IDXB[p].at[1]], GA[p].at[1], GAI[p]).wait()
            if edge_split:
                pltpu.make_async_copy(
                    ae_hbm.at[pl.ds(0, C)], AEV[p], GAE[p]).wait()
            else:
                pltpu.make_async_copy(
                    ae_hbm.at[pl.ds(0, C)],
                    AEV[p].at[pl.ds(0, C)], GAE[p]).wait()
                pltpu.make_async_copy(
                    ae_hbm.at[pl.ds(0, C)],
                    AEV[p].at[pl.ds(C, C)], GAE[p]).wait()
            @pl.when(ch >= 2)
            def _():
                pltpu.make_async_copy(MSG[p], acc.at[DSC[p]], SCAT[p]).wait()
            for t in range(C // 16):
                sl = pl.ds(t * 16, 16)
                DSC[p][sl] = DST[p][sl]
            fire_linear(ch + 2, p)

            def g_body(t, carry):
                sv0 = GA[p][0, pl.ds(16 * t, 16)] + AEV[p][pl.ds(16 * t, 16)]
                if not edge_split:
                    sv1 = (GA[p][1, pl.ds(16 * t, 16)]
                           + AEV[p][pl.ds(C + 16 * t, 16)])
                for m in range(16):
                    j = t * 16 + m
                    idxm = jnp.broadcast_to(jnp.int32(m), (16,))
                    rot0 = sv0.at[idxm].get(mode="promise_in_bounds")
                    if edge_split:
                        pair = rot0
                    else:
                        rot1 = sv1.at[idxm].get(mode="promise_in_bounds")
                        pair = jnp.where(iota < 1, rot0,
                                         jnp.where(iota < 2, rot1, 0.0))
                    tailv = ROWS[p][j, pl.ds(RW, 16)]
                    s = pair + tailv
                    e = jnp.where(s > 0, s, SLOPE * s)
                    ex = jnp.exp(e)
                    for fs in range(FV):
                        b = ex.at[jnp.broadcast_to(
                            jnp.int32((fs * 16) // HID), (16,))].get(
                            mode="promise_in_bounds")
                        MSG[p].at[j][pl.ds(fs * 16, 16)] = \
                            ROWS[p][j, pl.ds(fs * 16, 16)] * b
                    MSG[p].at[j][pl.ds(RW, 16)] = jnp.where(
                        iota < HH, ex, 0.0)
                return carry
            lax.fori_loop(0, C // 16, g_body, 0)

            pltpu.async_copy(MSG[p], acc.at[DSC[p]], SCAT[p], add=True)

        fire_linear(0, 0)
        fire_linear(1, 1)
        stage_a(0, 0)

        def pair(k, carry):
            ch0 = 2 * k
            stage_a(ch0 + 1, 1)
            stage_b(ch0, 0)
            @pl.when(ch0 + 1 < NCH)
            def _odd():
                stage_a(ch0 + 2, 0)
                stage_b(ch0 + 1, 1)
            return carry
        lax.fori_loop(0, NP, pair, 0)

        for p in range(2):
            pltpu.make_async_copy(MSG[p], acc.at[DSC[p]], SCAT[p]).wait()

        plsc.subcore_barrier()
        for k in range(-(-NZCH // 16)):
            zc = tid + 16 * k
            @pl.when(zc < NZCH)
            def _():
                r0 = zc * ZR
                pltpu.sync_copy(acc.at[pl.ds(r0, ZR)], zbuf)
                pltpu.sync_copy(zbuf, out_hbm.at[pl.ds(cid * N + r0, ZR)])

    return edge_pass


_edge_pass_l1 = _make_edge_pass(2, edge_split=False)
_edge_pass_l2 = _make_edge_pass(1, edge_split=True)


def kernel(x, edge_index, edge_attr, W1, We1, att1, proj1_w, proj1_b,
           W2, We2, att2, proj2_w, proj2_b):
    f32 = jnp.float32
    src = edge_index[0]
    dst = edge_index[1]

    W1cat = jnp.concatenate([W1[h] for h in range(HEADS)], axis=1)
    att = att1[:, :, 0]
    Ai = jnp.zeros((D_IN, HEADS), f32)
    Aj = jnp.zeros((D_IN, HEADS), f32)
    for h in range(HEADS):
        Ai = Ai.at[h * HID:(h + 1) * HID, h].set(att[h, :HID])
        Aj = Aj.at[h * HID:(h + 1) * HID, h].set(att[h, HID:2 * HID])
    A1 = jnp.concatenate([Ai, Aj], axis=1)
    Ve = jnp.stack([We1[h] @ att[h, 2 * HID:] for h in range(HEADS)], axis=1)
    ve2 = We2 @ att2[2 * HID:, 0]
    VeAll = jnp.concatenate(
        [Ve, ve2[:, None], jnp.zeros((D_EDGE, 3), f32)], axis=1)
    A2 = jnp.concatenate(
        [att2[:HID, :1], att2[HID:2 * HID, :1]], axis=1)

    BN = 1000
    nb = N // BN
    xwext1, ai8 = pl.pallas_call(
        _tc1_node_body,
        grid=(1,),
        in_specs=[pl.BlockSpec((N, D_IN), lambda i: (0, 0)),
                  pl.BlockSpec((D_IN, D_IN), lambda i: (0, 0)),
                  pl.BlockSpec((D_IN, 8), lambda i: (0, 0))],
        out_specs=[pl.BlockSpec((2 * N, 128), lambda i: (0, 0)),
                   pl.BlockSpec((8, N), lambda i: (0, 0))],
        out_shape=[jax.ShapeDtypeStruct((2 * N, 128), f32),
                   jax.ShapeDtypeStruct((8, N), f32)],
    )(x, W1cat, A1)

    BE = 2560
    ae8 = pl.pallas_call(
        _tc1_edge_body,
        grid=(E // BE,),
        in_specs=[pl.BlockSpec((BE, D_EDGE), lambda i: (i, 0)),
                  pl.BlockSpec((D_EDGE, 8), lambda i: (0, 0))],
        out_specs=pl.BlockSpec((8, BE), lambda i: (0, i)),
        out_shape=jax.ShapeDtypeStruct((8, E), f32),
    )(edge_attr, VeAll)

    ae8f = ae8.reshape(-1)
    part1 = _edge_pass_l1(src, dst, ai8.reshape(-1), ae8f,
                          xwext1)

    xw2ext, ai2 = pl.pallas_call(
        _tc2_body,
        grid=(nb,),
        in_specs=[pl.BlockSpec((BN, 128), lambda i: (i, 0)),
                  pl.BlockSpec((BN, 128), lambda i, _nb=nb: (i + _nb, 0)),
                  pl.BlockSpec((D_IN, D_IN), lambda i: (0, 0)),
                  pl.BlockSpec((1, D_IN), lambda i: (0, 0)),
                  pl.BlockSpec((D_IN, HID), lambda i: (0, 0)),
                  pl.BlockSpec((HID, 2), lambda i: (0, 0))],
        out_specs=[pl.BlockSpec((BN, 128), lambda i: (i, 0)),
                   pl.BlockSpec((BN, 1), lambda i: (i, 0))],
        out_shape=[jax.ShapeDtypeStruct((N, 128), f32),
                   jax.ShapeDtypeStruct((N, 1), f32)],
    )(part1, part1, proj1_w, proj1_b[None, :], W2, A2)

    part2 = _edge_pass_l2(src, dst, ai2.reshape(-1), ae8f,
                          xw2ext)

    importance = pl.pallas_call(
        _tc3_body,
        grid=(nb,),
        in_specs=[pl.BlockSpec((BN, 128), lambda i: (i, 0)),
                  pl.BlockSpec((BN, 128), lambda i, _nb=nb: (i + _nb, 0)),
                  pl.BlockSpec((HID, 1), lambda i: (0, 0)),
                  pl.BlockSpec((1, 1), lambda i: (0, 0))],
        out_specs=pl.BlockSpec((BN, 1), lambda i: (i, 0)),
        out_shape=jax.ShapeDtypeStruct((N, 1), f32),
    )(part2, part2, proj2_w, proj2_b[None, :])

    return importance


# speedup vs baseline: 40.6558x; 1.3506x over previous
import functools

import jax
import jax.numpy as jnp
from jax import lax
from jax.experimental import pallas as pl
from jax.experimental.pallas import tpu as pltpu
from jax.experimental.pallas import tpu_sc as plsc

N = 10000
E = 320000
D_IN = 128
HID = 32
HEADS = 4
D_EDGE = 16
SLOPE = 0.2

C = 80
ZR = 40
NZCH = N // ZR


def _elu(v):
    return jnp.where(v > 0, v, jnp.exp(v) - 1.0)


def _tc1_node_body(x_ref, w_ref, a_ref, xw_ref, ai_ref):
    xw = jnp.dot(x_ref[...], w_ref[...], preferred_element_type=jnp.float32)
    xw_ref[...] = xw
    ai_ref[...] = lax.dot_general(
        a_ref[...], xw, (((0,), (1,)), ((), ())),
        preferred_element_type=jnp.float32)


def _tc1_edge_body(ea_ref, ve_ref, ae8_ref):
    ae8_ref[...] = lax.dot_general(
        ve_ref[...], ea_ref[...], (((0,), (1,)), ((), ())),
        preferred_element_type=jnp.float32)


def _tc2_body(p0_ref, p1_ref, pw_ref, pb_ref, w2_ref, a2_ref,
              xw2ext_ref, a2t_ref):
    p0 = p0_ref[...]
    p1 = p1_ref[...]
    feats = []
    for h in range(HEADS):
        p = p0 if h < 2 else p1
        loc = h % 2
        num = p[:, HID * loc:HID * loc + HID]
        den = p[:, 2 * HID + loc:2 * HID + loc + 1] + 1e-16
        feats.append(num / den)
    out1 = jnp.concatenate(feats, axis=1)
    h = _elu(jnp.dot(out1, pw_ref[...], preferred_element_type=jnp.float32)
             + pb_ref[...])
    h = _elu(h)
    xw2 = jnp.dot(h, w2_ref[...], preferred_element_type=jnp.float32)
    bn = xw2.shape[0]
    xw2ext_ref[...] = jnp.concatenate(
        [xw2, jnp.zeros((bn, 96), jnp.float32)], axis=1)
    a2t_ref[...] = lax.dot_general(
        a2_ref[...], xw2, (((0,), (1,)), ((), ())),
        preferred_element_type=jnp.float32)


def _tc3_body(p0_ref, p1_ref, pw_ref, pb_ref, out_ref):
    acc = p0_ref[...] + p1_ref[...]
    out2 = acc[:, :HID] / (acc[:, HID:HID + 1] + 1e-16)
    out_ref[...] = _elu(
        jnp.dot(out2, pw_ref[...], preferred_element_type=jnp.float32)
        + pb_ref[...])


_MESH = plsc.VectorSubcoreMesh(core_axis_name="c", subcore_axis_name="s")


def _make_edge_pass(HH, edge_split):
    RW = HH * HID
    FV = RW // 16
    UW = FV + 1
    EPT = E // 16 if not edge_split else E // 32
    NCH = EPT // C
    NP = (NCH + 1) // 2

    @functools.partial(
        pl.kernel, mesh=_MESH,
        out_type=jax.ShapeDtypeStruct((2 * N, 128), jnp.float32),
        scratch_types=(
            [pltpu.VMEM((C,), jnp.int32) for _ in range(6)]
            + [pltpu.VMEM((4, C), jnp.int32) for _ in range(2)]
            + [pltpu.VMEM((4, C), jnp.float32) for _ in range(2)]
            + [pltpu.VMEM((C * HH,), jnp.float32) for _ in range(2)]
            + [pltpu.VMEM((C, 128), jnp.float32) for _ in range(4)]
            + [pltpu.VMEM((ZR, 128), jnp.float32)]
            + [pltpu.VMEM_SHARED((N, 128), jnp.float32)]
            + [pltpu.SemaphoreType.DMA for _ in range(12)]
        ),
    )
    def edge_pass(src_hbm, dst_hbm, ai_hbm, ae_hbm, xw_hbm, out_hbm,
                  srcv0, srcv1, dstv0, dstv1, dscat0, dscat1,
                  idx40, idx41, gbufA0, gbufA1, aev0, aev1,
                  rowsv0, rowsv1, msgv0, msgv1, zbuf, acc,
                  lsrc0, lsrc1, ldst0, ldst1, grow0, grow1,
                  gai0, gai1, gae0, gae1, scat0, scat1):
        SRC = [srcv0, srcv1]
        DST = [dstv0, dstv1]
        DSC = [dscat0, dscat1]
        IDX4 = [idx40, idx41]
        GA = [gbufA0, gbufA1]
        AEV = [aev0, aev1]
        ROWS = [rowsv0, rowsv1]
        MSG = [msgv0, msgv1]
        LSRC = [lsrc0, lsrc1]
        LDST = [ldst0, ldst1]
        GROW = [grow0, grow1]
        GAI = [gai0, gai1]
        GAE = [gae0, gae1]
        SCAT = [scat0, scat1]

        cid = lax.axis_index("c")
        tid = lax.axis_index("s")
        iota = lax.iota(jnp.int32, 16)
        zeros16 = jnp.zeros((16,), jnp.float32)

        def chunk_base(ch):
            if edge_split:
                return (tid * 2 + cid) * EPT + ch * C
            return tid * EPT + ch * C

        def zrow(t, _):
            r = t // 8
            k = t % 8
            zbuf.at[r][pl.ds(k * 16, 16)] = zeros16
            return _
        lax.fori_loop(0, ZR * 8, zrow, 0)

        def mpad(j, _):
            for p in range(2):
                for s in range(UW, 8):
                    MSG[p].at[j][pl.ds(s * 16, 16)] = zeros16
            return _
        lax.fori_loop(0, C, mpad, 0)

        for k in range(-(-NZCH // 16)):
            zc = tid + 16 * k
            @pl.when(zc < NZCH)
            def _():
                pltpu.sync_copy(zbuf, acc.at[pl.ds(zc * ZR, ZR)])
        plsc.subcore_barrier()

        def fire_linear(ch, p):
            @pl.when(ch < NCH)
            def _():
                base = chunk_base(ch)
                pltpu.async_copy(src_hbm.at[pl.ds(base, C)], SRC[p], LSRC[p])
                pltpu.async_copy(dst_hbm.at[pl.ds(base, C)], DST[p], LDST[p])

        def stage_a(ch, p):
            @pl.when(ch < NCH)
            def _():
                pltpu.make_async_copy(
                    src_hbm.at[pl.ds(0, C)], SRC[p], LSRC[p]).wait()
                pltpu.make_async_copy(
                    dst_hbm.at[pl.ds(0, C)], DST[p], LDST[p]).wait()
                base = chunk_base(ch)
                pltpu.async_copy(xw_hbm.at[SRC[p]], ROWS[p], GROW[p])
                if edge_split:
                    pltpu.async_copy(ae_hbm.at[pl.ds(4 * E + base, C)],
                                     AEV[p], GAE[p])
                    for t in range(C // 16):
                        sl = pl.ds(t * 16, 16)
                        IDX4[p].at[2][sl] = SRC[p][sl] + N
                    pltpu.async_copy(ai_hbm.at[DST[p]], GA[p].at[0], GAI[p])
                    pltpu.async_copy(ai_hbm.at[IDX4[p].at[2]],
                                     GA[p].at[2], GAI[p])
                else:
                    aoff = cid * (2 * E) + base
                    pltpu.async_copy(ae_hbm.at[pl.ds(aoff, C)],
                                     AEV[p].at[pl.ds(0, C)], GAE[p])
                    pltpu.async_copy(ae_hbm.at[pl.ds(aoff + E, C)],
                                     AEV[p].at[pl.ds(C, C)], GAE[p])
                    for t in range(C // 16):
                        sl = pl.ds(t * 16, 16)
                        IDX4[p].at[0][sl] = DST[p][sl] + cid * (2 * N)
                        IDX4[p].at[1][sl] = DST[p][sl] + cid * (2 * N) + N
                        IDX4[p].at[2][sl] = SRC[p][sl] + (4 + cid * 2) * N
                        IDX4[p].at[3][sl] = SRC[p][sl] + (5 + cid * 2) * N
                    for r in range(4):
                        pltpu.async_copy(ai_hbm.at[IDX4[p].at[r]],
                                         GA[p].at[r], GAI[p])

        def stage_b(ch, p):
            pltpu.make_async_copy(
                xw_hbm.at[SRC[p]], ROWS[p], GROW[p]).wait()
            if edge_split:
                pltpu.make_async_copy(
                    ai_hbm.at[DST[p]], GA[p].at[0], GAI[p]).wait()
                pltpu.make_async_copy(
                    ai_hbm.at[IDX4[p].at[2]], GA[p].at[2], GAI[p]).wait()
                pltpu.make_async_copy(
                    ae_hbm.at[pl.ds(0, C)], AEV[p], GAE[p]).wait()
            else:
                for r in range(4):
                    pltpu.make_async_copy(
                        ai_hbm.at[IDX4[p].at[r]], GA[p].at[r], GAI[p]).wait()
                pltpu.make_async_copy(
                    ae_hbm.at[pl.ds(0, C)],
                    AEV[p].at[pl.ds(0, C)], GAE[p]).wait()
                pltpu.make_async_copy(
                    ae_hbm.at[pl.ds(0, C)],
                    AEV[p].at[pl.ds(C, C)], GAE[p]).wait()
            @pl.when(ch >= 2)
            def _():
                pltpu.make_async_copy(MSG[p], acc.at[DSC[p]], SCAT[p]).wait()
            for t in range(C // 16):
                sl = pl.ds(t * 16, 16)
                DSC[p][sl] = DST[p][sl]
            fire_linear(ch + 2, p)

            col0 = 0 if edge_split else cid * 64

            def g_body(t, carry):
                sl = pl.ds(16 * t, 16)
                s0 = GA[p][0, sl] + GA[p][2, sl] + AEV[p][sl]
                ex0 = jnp.exp(jnp.maximum(s0, SLOPE * s0))
                if not edge_split:
                    s1 = (GA[p][1, sl] + GA[p][3, sl]
                          + AEV[p][pl.ds(C + 16 * t, 16)])
                    ex1 = jnp.exp(jnp.maximum(s1, SLOPE * s1))
                for m in range(16):
                    j = t * 16 + m
                    idxm = jnp.broadcast_to(jnp.int32(m), (16,))
                    b0 = ex0.at[idxm].get(mode="promise_in_bounds")
                    if not edge_split:
                        b1 = ex1.at[idxm].get(mode="promise_in_bounds")
                        tl = jnp.where(iota < 1, b0,
                                       jnp.where(iota < 2, b1, 0.0))
                    else:
                        tl = jnp.where(iota < 1, b0, 0.0)
                    for fs in range(FV):
                        b = b0 if (fs * 16) // HID == 0 else b1
                        MSG[p].at[j][pl.ds(fs * 16, 16)] = \
                            ROWS[p][j, pl.ds(col0 + fs * 16, 16)] * b
                    MSG[p].at[j][pl.ds(RW, 16)] = tl
                return carry
            lax.fori_loop(0, C // 16, g_body, 0)

            pltpu.async_copy(MSG[p], acc.at[DSC[p]], SCAT[p], add=True)

        fire_linear(0, 0)
        fire_linear(1, 1)
        stage_a(0, 0)

        def pair(k, carry):
            ch0 = 2 * k
            stage_a(ch0 + 1, 1)
            stage_b(ch0, 0)
            @pl.when(ch0 + 1 < NCH)
            def _odd():
                stage_a(ch0 + 2, 0)
                stage_b(ch0 + 1, 1)
            return carry
        lax.fori_loop(0, NP, pair, 0)

        for p in range(2):
            pltpu.make_async_copy(MSG[p], acc.at[DSC[p]], SCAT[p]).wait()

        plsc.subcore_barrier()
        for k in range(-(-NZCH // 16)):
            zc = tid + 16 * k
            @pl.when(zc < NZCH)
            def _():
                r0 = zc * ZR
                pltpu.sync_copy(acc.at[pl.ds(r0, ZR)], zbuf)
                pltpu.sync_copy(zbuf, out_hbm.at[pl.ds(cid * N + r0, ZR)])

    return edge_pass


_edge_pass_l1 = _make_edge_pass(2, edge_split=False)
_edge_pass_l2 = _make_edge_pass(1, edge_split=True)


def kernel(x, edge_index, edge_attr, W1, We1, att1, proj1_w, proj1_b,
           W2, We2, att2, proj2_w, proj2_b):
    f32 = jnp.float32
    src = edge_index[0]
    dst = edge_index[1]

    W1cat = jnp.concatenate([W1[h] for h in range(HEADS)], axis=1)
    att = att1[:, :, 0]
    Ai = jnp.zeros((D_IN, HEADS), f32)
    Aj = jnp.zeros((D_IN, HEADS), f32)
    for h in range(HEADS):
        Ai = Ai.at[h * HID:(h + 1) * HID, h].set(att[h, :HID])
        Aj = Aj.at[h * HID:(h + 1) * HID, h].set(att[h, HID:2 * HID])
    A1 = jnp.concatenate([Ai, Aj], axis=1)
    Ve = jnp.stack([We1[h] @ att[h, 2 * HID:] for h in range(HEADS)], axis=1)
    ve2 = We2 @ att2[2 * HID:, 0]
    VeAll = jnp.concatenate(
        [Ve, ve2[:, None], jnp.zeros((D_EDGE, 3), f32)], axis=1)
    A2 = jnp.concatenate(
        [att2[:HID, :1], att2[HID:2 * HID, :1]], axis=1)

    BN = 1000
    nb = N // BN
    xw1, ai8 = pl.pallas_call(
        _tc1_node_body,
        grid=(1,),
        in_specs=[pl.BlockSpec((N, D_IN), lambda i: (0, 0)),
                  pl.BlockSpec((D_IN, D_IN), lambda i: (0, 0)),
                  pl.BlockSpec((D_IN, 8), lambda i: (0, 0))],
        out_specs=[pl.BlockSpec((N, 128), lambda i: (0, 0)),
                   pl.BlockSpec((8, N), lambda i: (0, 0))],
        out_shape=[jax.ShapeDtypeStruct((N, 128), f32),
                   jax.ShapeDtypeStruct((8, N), f32)],
    )(x, W1cat, A1)

    BE = 2560
    ae8 = pl.pallas_call(
        _tc1_edge_body,
        grid=(E // BE,),
        in_specs=[pl.BlockSpec((BE, D_EDGE), lambda i: (i, 0)),
                  pl.BlockSpec((D_EDGE, 8), lambda i: (0, 0))],
        out_specs=pl.BlockSpec((8, BE), lambda i: (0, i)),
        out_shape=jax.ShapeDtypeStruct((8, E), f32),
    )(edge_attr, VeAll)

    ae8f = ae8.reshape(-1)
    part1 = _edge_pass_l1(src, dst, ai8.reshape(-1), ae8f,
                          xw1)

    xw2ext, a2t = pl.pallas_call(
        _tc2_body,
        grid=(1,),
        in_specs=[pl.BlockSpec((N, 128), lambda i: (0, 0)),
                  pl.BlockSpec((N, 128), lambda i: (1, 0)),
                  pl.BlockSpec((D_IN, D_IN), lambda i: (0, 0)),
                  pl.BlockSpec((1, D_IN), lambda i: (0, 0)),
                  pl.BlockSpec((D_IN, HID), lambda i: (0, 0)),
                  pl.BlockSpec((HID, 2), lambda i: (0, 0))],
        out_specs=[pl.BlockSpec((N, 128), lambda i: (0, 0)),
                   pl.BlockSpec((2, N), lambda i: (0, 0))],
        out_shape=[jax.ShapeDtypeStruct((N, 128), f32),
                   jax.ShapeDtypeStruct((2, N), f32)],
    )(part1, part1, proj1_w, proj1_b[None, :], W2, A2)

    part2 = _edge_pass_l2(src, dst, a2t.reshape(-1), ae8f,
                          xw2ext)

    importance = pl.pallas_call(
        _tc3_body,
        grid=(nb,),
        in_specs=[pl.BlockSpec((BN, 128), lambda i: (i, 0)),
                  pl.BlockSpec((BN, 128), lambda i, _nb=nb: (i + _nb, 0)),
                  pl.BlockSpec((HID, 1), lambda i: (0, 0)),
                  pl.BlockSpec((1, 1), lambda i: (0, 0))],
        out_specs=pl.BlockSpec((BN, 1), lambda i: (i, 0)),
        out_shape=jax.ShapeDtypeStruct((N, 1), f32),
    )(part2, part2, proj2_w, proj2_b[None, :])

    return importance


# speedup vs baseline: 40.8487x; 1.0047x over previous
import functools

import jax
import jax.numpy as jnp
from jax import lax
from jax.experimental import pallas as pl
from jax.experimental.pallas import tpu as pltpu
from jax.experimental.pallas import tpu_sc as plsc

N = 10000
E = 320000
D_IN = 128
HID = 32
HEADS = 4
D_EDGE = 16
SLOPE = 0.2

C = 80
ZR = 40
NZCH = N // ZR


def _elu(v):
    return jnp.where(v > 0, v, jnp.exp(v) - 1.0)


def _tc1_body(x_ref, w_ref, a_ref, ea_ref, ve_ref, xw_ref, ai_ref, ae8_ref):
    @pl.when(pl.program_id(0) == 0)
    def _():
        xw = jnp.dot(x_ref[...], w_ref[...],
                     preferred_element_type=jnp.float32)
        xw_ref[...] = xw
        ai_ref[...] = lax.dot_general(
            a_ref[...], xw, (((0,), (1,)), ((), ())),
            preferred_element_type=jnp.float32)
    ae8_ref[...] = lax.dot_general(
        ve_ref[...], ea_ref[...], (((0,), (1,)), ((), ())),
        preferred_element_type=jnp.float32)


def _tc2_body(p0_ref, p1_ref, pw_ref, pb_ref, w2_ref, a2_ref,
              xw2ext_ref, a2t_ref):
    p0 = p0_ref[...]
    p1 = p1_ref[...]
    feats = []
    for h in range(HEADS):
        p = p0 if h < 2 else p1
        loc = h % 2
        num = p[:, HID * loc:HID * loc + HID]
        den = p[:, 2 * HID + loc:2 * HID + loc + 1] + 1e-16
        feats.append(num / den)
    out1 = jnp.concatenate(feats, axis=1)
    h = _elu(jnp.dot(out1, pw_ref[...], preferred_element_type=jnp.float32)
             + pb_ref[...])
    h = _elu(h)
    xw2 = jnp.dot(h, w2_ref[...], preferred_element_type=jnp.float32)
    bn = xw2.shape[0]
    xw2ext_ref[...] = jnp.concatenate(
        [xw2, jnp.zeros((bn, 96), jnp.float32)], axis=1)
    a2t_ref[...] = lax.dot_general(
        a2_ref[...], xw2, (((0,), (1,)), ((), ())),
        preferred_element_type=jnp.float32)


def _tc3_body(p0_ref, p1_ref, pw_ref, pb_ref, out_ref):
    acc = p0_ref[...] + p1_ref[...]
    out2 = acc[:, :HID] / (acc[:, HID:HID + 1] + 1e-16)
    out_ref[...] = _elu(
        jnp.dot(out2, pw_ref[...], preferred_element_type=jnp.float32)
        + pb_ref[...])


_MESH = plsc.VectorSubcoreMesh(core_axis_name="c", subcore_axis_name="s")


def _make_edge_pass(HH, edge_split):
    RW = HH * HID
    FV = RW // 16
    UW = FV + 1
    EPT = E // 16 if not edge_split else E // 32
    NCH = EPT // C
    NP = (NCH + 1) // 2

    @functools.partial(
        pl.kernel, mesh=_MESH,
        out_type=jax.ShapeDtypeStruct((2 * N, 128), jnp.float32),
        scratch_types=(
            [pltpu.VMEM((C,), jnp.int32) for _ in range(6)]
            + [pltpu.VMEM((4, C), jnp.int32) for _ in range(2)]
            + [pltpu.VMEM((4, C), jnp.float32) for _ in range(2)]
            + [pltpu.VMEM((C * HH,), jnp.float32) for _ in range(2)]
            + [pltpu.VMEM((C, 128), jnp.float32) for _ in range(4)]
            + [pltpu.VMEM((ZR, 128), jnp.float32)]
            + [pltpu.VMEM_SHARED((N, 128), jnp.float32)]
            + [pltpu.SemaphoreType.DMA for _ in range(12)]
        ),
    )
    def edge_pass(src_hbm, dst_hbm, ai_hbm, ae_hbm, xw_hbm, out_hbm,
                  srcv0, srcv1, dstv0, dstv1, dscat0, dscat1,
                  idx40, idx41, gbufA0, gbufA1, aev0, aev1,
                  rowsv0, rowsv1, msgv0, msgv1, zbuf, acc,
                  lsrc0, lsrc1, ldst0, ldst1, grow0, grow1,
                  gai0, gai1, gae0, gae1, scat0, scat1):
        SRC = [srcv0, srcv1]
        DST = [dstv0, dstv1]
        DSC = [dscat0, dscat1]
        IDX4 = [idx40, idx41]
        GA = [gbufA0, gbufA1]
        AEV = [aev0, aev1]
        ROWS = [rowsv0, rowsv1]
        MSG = [msgv0, msgv1]
        LSRC = [lsrc0, lsrc1]
        LDST = [ldst0, ldst1]
        GROW = [grow0, grow1]
        GAI = [gai0, gai1]
        GAE = [gae0, gae1]
        SCAT = [scat0, scat1]

        cid = lax.axis_index("c")
        tid = lax.axis_index("s")
        iota = lax.iota(jnp.int32, 16)
        zeros16 = jnp.zeros((16,), jnp.float32)

        def chunk_base(ch):
            if edge_split:
                return (tid * 2 + cid) * EPT + ch * C
            return tid * EPT + ch * C

        def zrow(t, _):
            r = t // 8
            k = t % 8
            zbuf.at[r][pl.ds(k * 16, 16)] = zeros16
            return _
        lax.fori_loop(0, ZR * 8, zrow, 0)

        def mpad(j, _):
            for p in range(2):
                for s in range(UW, 8):
                    MSG[p].at[j][pl.ds(s * 16, 16)] = zeros16
            return _
        lax.fori_loop(0, C, mpad, 0)

        for k in range(-(-NZCH // 16)):
            zc = tid + 16 * k
            @pl.when(zc < NZCH)
            def _():
                pltpu.sync_copy(zbuf, acc.at[pl.ds(zc * ZR, ZR)])
        plsc.subcore_barrier()

        def fire_linear(ch, p):
            @pl.when(ch < NCH)
            def _():
                base = chunk_base(ch)
                pltpu.async_copy(src_hbm.at[pl.ds(base, C)], SRC[p], LSRC[p])
                pltpu.async_copy(dst_hbm.at[pl.ds(base, C)], DST[p], LDST[p])

        def stage_a(ch, p):
            @pl.when(ch < NCH)
            def _():
                pltpu.make_async_copy(
                    src_hbm.at[pl.ds(0, C)], SRC[p], LSRC[p]).wait()
                pltpu.make_async_copy(
                    dst_hbm.at[pl.ds(0, C)], DST[p], LDST[p]).wait()
                base = chunk_base(ch)
                pltpu.async_copy(xw_hbm.at[SRC[p]], ROWS[p], GROW[p])
                if edge_split:
                    pltpu.async_copy(ae_hbm.at[pl.ds(4 * E + base, C)],
                                     AEV[p], GAE[p])
                    for t in range(C // 16):
                        sl = pl.ds(t * 16, 16)
                        IDX4[p].at[2][sl] = SRC[p][sl] + N
                    pltpu.async_copy(ai_hbm.at[DST[p]], GA[p].at[0], GAI[p])
                    pltpu.async_copy(ai_hbm.at[IDX4[p].at[2]],
                                     GA[p].at[2], GAI[p])
                else:
                    aoff = cid * (2 * E) + base
                    pltpu.async_copy(ae_hbm.at[pl.ds(aoff, C)],
                                     AEV[p].at[pl.ds(0, C)], GAE[p])
                    pltpu.async_copy(ae_hbm.at[pl.ds(aoff + E, C)],
                                     AEV[p].at[pl.ds(C, C)], GAE[p])
                    for t in range(C // 16):
                        sl = pl.ds(t * 16, 16)
                        IDX4[p].at[0][sl] = DST[p][sl] + cid * (2 * N)
                        IDX4[p].at[1][sl] = DST[p][sl] + cid * (2 * N) + N
                        IDX4[p].at[2][sl] = SRC[p][sl] + (4 + cid * 2) * N
                        IDX4[p].at[3][sl] = SRC[p][sl] + (5 + cid * 2) * N
                    for r in range(4):
                        pltpu.async_copy(ai_hbm.at[IDX4[p].at[r]],
                                         GA[p].at[r], GAI[p])

        def stage_b(ch, p):
            pltpu.make_async_copy(
                xw_hbm.at[SRC[p]], ROWS[p], GROW[p]).wait()
            if edge_split:
                pltpu.make_async_copy(
                    ai_hbm.at[DST[p]], GA[p].at[0], GAI[p]).wait()
                pltpu.make_async_copy(
                    ai_hbm.at[IDX4[p].at[2]], GA[p].at[2], GAI[p]).wait()
                pltpu.make_async_copy(
                    ae_hbm.at[pl.ds(0, C)], AEV[p], GAE[p]).wait()
            else:
                for r in range(4):
                    pltpu.make_async_copy(
                        ai_hbm.at[IDX4[p].at[r]], GA[p].at[r], GAI[p]).wait()
                pltpu.make_async_copy(
                    ae_hbm.at[pl.ds(0, C)],
                    AEV[p].at[pl.ds(0, C)], GAE[p]).wait()
                pltpu.make_async_copy(
                    ae_hbm.at[pl.ds(0, C)],
                    AEV[p].at[pl.ds(C, C)], GAE[p]).wait()
            @pl.when(ch >= 2)
            def _():
                pltpu.make_async_copy(MSG[p], acc.at[DSC[p]], SCAT[p]).wait()
            for t in range(C // 16):
                sl = pl.ds(t * 16, 16)
                DSC[p][sl] = DST[p][sl]
            fire_linear(ch + 2, p)

            col0 = 0 if edge_split else cid * 64

            def g_body(t, carry):
                sl = pl.ds(16 * t, 16)
                s0 = GA[p][0, sl] + GA[p][2, sl] + AEV[p][sl]
                ex0 = jnp.exp(jnp.maximum(s0, SLOPE * s0))
                if not edge_split:
                    s1 = (GA[p][1, sl] + GA[p][3, sl]
                          + AEV[p][pl.ds(C + 16 * t, 16)])
                    ex1 = jnp.exp(jnp.maximum(s1, SLOPE * s1))
                for m in range(16):
                    j = t * 16 + m
                    idxm = jnp.broadcast_to(jnp.int32(m), (16,))
                    b0 = ex0.at[idxm].get(mode="promise_in_bounds")
                    if not edge_split:
                        b1 = ex1.at[idxm].get(mode="promise_in_bounds")
                        tl = jnp.where(iota < 1, b0,
                                       jnp.where(iota < 2, b1, 0.0))
                    else:
                        tl = jnp.where(iota < 1, b0, 0.0)
                    for fs in range(FV):
                        b = b0 if (fs * 16) // HID == 0 else b1
                        MSG[p].at[j][pl.ds(fs * 16, 16)] = \
                            ROWS[p][j, pl.ds(col0 + fs * 16, 16)] * b
                    MSG[p].at[j][pl.ds(RW, 16)] = tl
                return carry
            lax.fori_loop(0, C // 16, g_body, 0)

            pltpu.async_copy(MSG[p], acc.at[DSC[p]], SCAT[p], add=True)

        fire_linear(0, 0)
        fire_linear(1, 1)
        stage_a(0, 0)

        def pair(k, carry):
            ch0 = 2 * k
            stage_a(ch0 + 1, 1)
            stage_b(ch0, 0)
            @pl.when(ch0 + 1 < NCH)
            def _odd():
                stage_a(ch0 + 2, 0)
                stage_b(ch0 + 1, 1)
            return carry
        lax.fori_loop(0, NP, pair, 0)

        for p in range(2):
            pltpu.make_async_copy(MSG[p], acc.at[DSC[p]], SCAT[p]).wait()

        plsc.subcore_barrier()
        for k in range(-(-NZCH // 16)):
            zc = tid + 16 * k
            @pl.when(zc < NZCH)
            def _():
                r0 = zc * ZR
                pltpu.sync_copy(acc.at[pl.ds(r0, ZR)], zbuf)
                pltpu.sync_copy(zbuf, out_hbm.at[pl.ds(cid * N + r0, ZR)])

    return edge_pass


_edge_pass_l1 = _make_edge_pass(2, edge_split=False)
_edge_pass_l2 = _make_edge_pass(1, edge_split=True)


def kernel(x, edge_index, edge_attr, W1, We1, att1, proj1_w, proj1_b,
           W2, We2, att2, proj2_w, proj2_b):
    f32 = jnp.float32
    src = edge_index[0]
    dst = edge_index[1]

    W1cat = jnp.concatenate([W1[h] for h in range(HEADS)], axis=1)
    att = att1[:, :, 0]
    Ai = jnp.zeros((D_IN, HEADS), f32)
    Aj = jnp.zeros((D_IN, HEADS), f32)
    for h in range(HEADS):
        Ai = Ai.at[h * HID:(h + 1) * HID, h].set(att[h, :HID])
        Aj = Aj.at[h * HID:(h + 1) * HID, h].set(att[h, HID:2 * HID])
    A1 = jnp.concatenate([Ai, Aj], axis=1)
    Ve = jnp.stack([We1[h] @ att[h, 2 * HID:] for h in range(HEADS)], axis=1)
    ve2 = We2 @ att2[2 * HID:, 0]
    VeAll = jnp.concatenate(
        [Ve, ve2[:, None], jnp.zeros((D_EDGE, 3), f32)], axis=1)
    A2 = jnp.concatenate(
        [att2[:HID, :1], att2[HID:2 * HID, :1]], axis=1)

    BN = 1000
    nb = N // BN
    BE = 2560
    xw1, ai8, ae8 = pl.pallas_call(
        _tc1_body,
        grid=(E // BE,),
        in_specs=[pl.BlockSpec((N, D_IN), lambda i: (0, 0)),
                  pl.BlockSpec((D_IN, D_IN), lambda i: (0, 0)),
                  pl.BlockSpec((D_IN, 8), lambda i: (0, 0)),
                  pl.BlockSpec((BE, D_EDGE), lambda i: (i, 0)),
                  pl.BlockSpec((D_EDGE, 8), lambda i: (0, 0))],
        out_specs=[pl.BlockSpec((N, 128), lambda i: (0, 0)),
                   pl.BlockSpec((8, N), lambda i: (0, 0)),
                   pl.BlockSpec((8, BE), lambda i: (0, i))],
        out_shape=[jax.ShapeDtypeStruct((N, 128), f32),
                   jax.ShapeDtypeStruct((8, N), f32),
                   jax.ShapeDtypeStruct((8, E), f32)],
    )(x, W1cat, A1, edge_attr, VeAll)

    ae8f = ae8.reshape(-1)
    part1 = _edge_pass_l1(src, dst, ai8.reshape(-1), ae8f,
                          xw1)

    xw2ext, a2t = pl.pallas_call(
        _tc2_body,
        grid=(1,),
        in_specs=[pl.BlockSpec((N, 128), lambda i: (0, 0)),
                  pl.BlockSpec((N, 128), lambda i: (1, 0)),
                  pl.BlockSpec((D_IN, D_IN), lambda i: (0, 0)),
                  pl.BlockSpec((1, D_IN), lambda i: (0, 0)),
                  pl.BlockSpec((D_IN, HID), lambda i: (0, 0)),
                  pl.BlockSpec((HID, 2), lambda i: (0, 0))],
        out_specs=[pl.BlockSpec((N, 128), lambda i: (0, 0)),
                   pl.BlockSpec((2, N), lambda i: (0, 0))],
        out_shape=[jax.ShapeDtypeStruct((N, 128), f32),
                   jax.ShapeDtypeStruct((2, N), f32)],
    )(part1, part1, proj1_w, proj1_b[None, :], W2, A2)

    part2 = _edge_pass_l2(src, dst, a2t.reshape(-1), ae8f,
                          xw2ext)

    importance = pl.pallas_call(
        _tc3_body,
        grid=(nb,),
        in_specs=[pl.BlockSpec((BN, 128), lambda i: (i, 0)),
                  pl.BlockSpec((BN, 128), lambda i, _nb=nb: (i + _nb, 0)),
                  pl.BlockSpec((HID, 1), lambda i: (0, 0)),
                  pl.BlockSpec((1, 1), lambda i: (0, 0))],
        out_specs=pl.BlockSpec((BN, 1), lambda i: (i, 0)),
        out_shape=jax.ShapeDtypeStruct((N, 1), f32),
    )(part2, part2, proj2_w, proj2_b[None, :])

    return importance
